# initial kernel scaffold (unmeasured)
import jax
import jax.numpy as jnp
from jax import lax
from jax.experimental import pallas as pl
from jax.experimental.pallas import tpu as pltpu

N_DEV = 16


def kernel(A, B):
    m_per, k = A.shape
    _, n = B.shape

    def body(a_ref, b_ref, out_ref, ag_ref, send_sems, recv_sems):
        my = lax.axis_index("i")
        left = lax.rem(my + N_DEV - 1, N_DEV)
        right = lax.rem(my + 1, N_DEV)

        barrier_sem = pltpu.get_barrier_semaphore()
        for nbr in (left, right):
            pl.semaphore_signal(
                barrier_sem, inc=1,
                device_id=(nbr,), device_id_type=pl.DeviceIdType.MESH,
            )
        pl.semaphore_wait(barrier_sem, 2)

        b_bf = b_ref[...].astype(jnp.bfloat16)
        my_a = a_ref[...].astype(jnp.bfloat16)
        ag_ref[pl.ds(my * m_per, m_per), :] = my_a
        out_ref[pl.ds(my * m_per, m_per), :] = jnp.dot(
            my_a, b_bf, preferred_element_type=jnp.float32
        ).astype(jnp.bfloat16)

        for h in range(N_DEV - 1):
            origin = lax.rem(my + N_DEV - h, N_DEV)
            incoming = lax.rem(my + 2 * N_DEV - h - 1, N_DEV)

            send = pltpu.make_async_remote_copy(
                src_ref=ag_ref.at[pl.ds(origin * m_per, m_per), :],
                dst_ref=ag_ref.at[pl.ds(origin * m_per, m_per), :],
                send_sem=send_sems.at[h],
                recv_sem=recv_sems.at[h],
                device_id=(right,),
                device_id_type=pl.DeviceIdType.MESH,
            )
            send.start()
            recv = pltpu.make_async_remote_copy(
                src_ref=ag_ref.at[pl.ds(incoming * m_per, m_per), :],
                dst_ref=ag_ref.at[pl.ds(incoming * m_per, m_per), :],
                send_sem=send_sems.at[h],
                recv_sem=recv_sems.at[h],
                device_id=(left,),
                device_id_type=pl.DeviceIdType.MESH,
            )
            send.wait_send()
            recv.wait_recv()

            a_in = ag_ref[pl.ds(incoming * m_per, m_per), :]
            out_ref[pl.ds(incoming * m_per, m_per), :] = jnp.dot(
                a_in, b_bf, preferred_element_type=jnp.float32
            ).astype(jnp.bfloat16)

    return pl.pallas_call(
        body,
        out_shape=jax.ShapeDtypeStruct((N_DEV * m_per, n), jnp.bfloat16),
        in_specs=[
            pl.BlockSpec(memory_space=pltpu.VMEM),
            pl.BlockSpec(memory_space=pltpu.VMEM),
        ],
        out_specs=pl.BlockSpec(memory_space=pltpu.VMEM),
        scratch_shapes=[
            pltpu.VMEM((N_DEV * m_per, k), jnp.bfloat16),
            pltpu.SemaphoreType.DMA((N_DEV - 1,)),
            pltpu.SemaphoreType.DMA((N_DEV - 1,)),
        ],
        compiler_params=pltpu.CompilerParams(collective_id=0),
    )(A, B)


# baseline (device time: 227313 ns/iter reference)
import jax
import jax.numpy as jnp
from jax import lax
from jax.experimental import pallas as pl
from jax.experimental.pallas import tpu as pltpu

N_DEV = 16


def kernel(A, B):
    m_per, k = A.shape
    _, n = B.shape
    A = A.astype(jnp.bfloat16)
    B = B.astype(jnp.bfloat16)

    def body(a_ref, b_ref, out_ref, ag_ref, stage_ref,
             send_sems, recv_sems, copy_sem):
        my = lax.axis_index("i")
        left = lax.rem(my + N_DEV - 1, N_DEV)
        right = lax.rem(my + 1, N_DEV)

        barrier_sem = pltpu.get_barrier_semaphore()
        for nbr in (left, right):
            pl.semaphore_signal(
                barrier_sem, inc=1,
                device_id=(nbr,), device_id_type=pl.DeviceIdType.MESH,
            )
        pl.semaphore_wait(barrier_sem, 2)

        ag_ref[pl.ds(my * m_per, m_per), :] = a_ref[...]

        def compute_chunk(idx):
            stage_ref[...] = jnp.dot(
                ag_ref[pl.ds(idx * m_per, m_per), :], b_ref[...],
                preferred_element_type=jnp.float32,
            ).astype(jnp.bfloat16)
            copy = pltpu.make_async_copy(
                stage_ref, out_ref.at[pl.ds(idx * m_per, m_per), :], copy_sem
            )
            copy.start()
            copy.wait()

        for h in range(N_DEV - 1):
            origin = lax.rem(my + N_DEV - h, N_DEV)
            incoming = lax.rem(my + 2 * N_DEV - h - 1, N_DEV)

            send = pltpu.make_async_remote_copy(
                src_ref=ag_ref.at[pl.ds(origin * m_per, m_per), :],
                dst_ref=ag_ref.at[pl.ds(origin * m_per, m_per), :],
                send_sem=send_sems.at[h],
                recv_sem=recv_sems.at[h],
                device_id=(right,),
                device_id_type=pl.DeviceIdType.MESH,
            )
            send.start()
            compute_chunk(origin)
            recv = pltpu.make_async_remote_copy(
                src_ref=ag_ref.at[pl.ds(incoming * m_per, m_per), :],
                dst_ref=ag_ref.at[pl.ds(incoming * m_per, m_per), :],
                send_sem=send_sems.at[h],
                recv_sem=recv_sems.at[h],
                device_id=(left,),
                device_id_type=pl.DeviceIdType.MESH,
            )
            send.wait_send()
            recv.wait_recv()

        compute_chunk(lax.rem(my + N_DEV + 1, N_DEV))

    return pl.pallas_call(
        body,
        out_shape=jax.ShapeDtypeStruct((N_DEV * m_per, n), jnp.bfloat16),
        in_specs=[
            pl.BlockSpec(memory_space=pltpu.VMEM),
            pl.BlockSpec(memory_space=pltpu.VMEM),
        ],
        out_specs=pl.BlockSpec(memory_space=pl.ANY),
        scratch_shapes=[
            pltpu.VMEM((N_DEV * m_per, k), jnp.bfloat16),
            pltpu.VMEM((m_per, n), jnp.bfloat16),
            pltpu.SemaphoreType.DMA((N_DEV - 1,)),
            pltpu.SemaphoreType.DMA((N_DEV - 1,)),
            pltpu.SemaphoreType.DMA,
        ],
        compiler_params=pltpu.CompilerParams(collective_id=0),
    )(A, B)


# device time: 148199 ns/iter; 1.5338x vs baseline; 1.5338x over previous
import jax
import jax.numpy as jnp
from jax import lax
from jax.experimental import pallas as pl
from jax.experimental.pallas import tpu as pltpu

N_DEV = 16


def kernel(A, B):
    m_per, k = A.shape
    _, n = B.shape
    A = A.astype(jnp.bfloat16)
    B = B.astype(jnp.bfloat16)

    CW = 8
    CCW = N_DEV - 1 - CW

    def body(a_ref, b_ref, out_ref, ag_ref, stage_ref,
             cw_send_sems, cw_recv_sems, ccw_send_sems, ccw_recv_sems,
             copy_sem):
        my = lax.axis_index("i")
        left = lax.rem(my + N_DEV - 1, N_DEV)
        right = lax.rem(my + 1, N_DEV)

        barrier_sem = pltpu.get_barrier_semaphore()
        for nbr in (left, right):
            pl.semaphore_signal(
                barrier_sem, inc=1,
                device_id=(nbr,), device_id_type=pl.DeviceIdType.MESH,
            )
        pl.semaphore_wait(barrier_sem, 2)

        ag_ref[pl.ds(my * m_per, m_per), :] = a_ref[...]

        def compute_chunk(idx):
            stage_ref[...] = jnp.dot(
                ag_ref[pl.ds(idx * m_per, m_per), :], b_ref[...],
                preferred_element_type=jnp.float32,
            ).astype(jnp.bfloat16)
            copy = pltpu.make_async_copy(
                stage_ref, out_ref.at[pl.ds(idx * m_per, m_per), :], copy_sem
            )
            copy.start()
            copy.wait()

        def chunk_rdma(idx, send_sem, recv_sem, dev):
            return pltpu.make_async_remote_copy(
                src_ref=ag_ref.at[pl.ds(idx * m_per, m_per), :],
                dst_ref=ag_ref.at[pl.ds(idx * m_per, m_per), :],
                send_sem=send_sem,
                recv_sem=recv_sem,
                device_id=(dev,),
                device_id_type=pl.DeviceIdType.MESH,
            )

        for h in range(CW):
            cw_origin = lax.rem(my + N_DEV - h, N_DEV)
            cw_in = lax.rem(my + 2 * N_DEV - h - 1, N_DEV)
            send_cw = chunk_rdma(
                cw_origin, cw_send_sems.at[h], cw_recv_sems.at[h], right
            )
            send_cw.start()
            if h < CCW:
                ccw_origin = lax.rem(my + h, N_DEV)
                ccw_in = lax.rem(my + h + 1, N_DEV)
                send_ccw = chunk_rdma(
                    ccw_origin, ccw_send_sems.at[h], ccw_recv_sems.at[h], left
                )
                send_ccw.start()
            if h == 0:
                compute_chunk(my)
            else:
                compute_chunk(lax.rem(my + N_DEV - h, N_DEV))
                compute_chunk(lax.rem(my + h, N_DEV))
            send_cw.wait_send()
            chunk_rdma(
                cw_in, cw_send_sems.at[h], cw_recv_sems.at[h], left
            ).wait_recv()
            if h < CCW:
                send_ccw.wait_send()
                chunk_rdma(
                    ccw_in, ccw_send_sems.at[h], ccw_recv_sems.at[h], right
                ).wait_recv()

        compute_chunk(lax.rem(my + N_DEV - CW, N_DEV))

    return pl.pallas_call(
        body,
        out_shape=jax.ShapeDtypeStruct((N_DEV * m_per, n), jnp.bfloat16),
        in_specs=[
            pl.BlockSpec(memory_space=pltpu.VMEM),
            pl.BlockSpec(memory_space=pltpu.VMEM),
        ],
        out_specs=pl.BlockSpec(memory_space=pl.ANY),
        scratch_shapes=[
            pltpu.VMEM((N_DEV * m_per, k), jnp.bfloat16),
            pltpu.VMEM((m_per, n), jnp.bfloat16),
            pltpu.SemaphoreType.DMA((CW,)),
            pltpu.SemaphoreType.DMA((CW,)),
            pltpu.SemaphoreType.DMA((CCW,)),
            pltpu.SemaphoreType.DMA((CCW,)),
            pltpu.SemaphoreType.DMA,
        ],
        compiler_params=pltpu.CompilerParams(collective_id=0),
    )(A, B)


# device time: 136049 ns/iter; 1.6708x vs baseline; 1.0893x over previous
import jax
import jax.numpy as jnp
from jax import lax
from jax.experimental import pallas as pl
from jax.experimental.pallas import tpu as pltpu

N_DEV = 16


def kernel(A, B):
    m_per, k = A.shape
    _, n = B.shape
    A = A.astype(jnp.bfloat16)
    B = B.astype(jnp.bfloat16)

    CW = 8
    CCW = N_DEV - 1 - CW

    RING = (0, 4, 8, 12, 15, 11, 7, 3, 2, 6, 10, 14, 13, 9, 5, 1)
    POS = tuple(RING.index(i) for i in range(N_DEV))

    def body(a_ref, b_ref, out_ref, ag_ref, stage_ref,
             cw_send_sems, cw_recv_sems, ccw_send_sems, ccw_recv_sems,
             copy_sem):
        my = lax.axis_index("i")

        def lut(table, idx):
            r = jnp.int32(table[0])
            for i in range(1, N_DEV):
                r = jnp.where(idx == i, jnp.int32(table[i]), r)
            return r

        p = lut(POS, my)
        left = lut(RING, lax.rem(p + N_DEV - 1, N_DEV))
        right = lut(RING, lax.rem(p + 1, N_DEV))

        barrier_sem = pltpu.get_barrier_semaphore()
        for nbr in (left, right):
            pl.semaphore_signal(
                barrier_sem, inc=1,
                device_id=(nbr,), device_id_type=pl.DeviceIdType.MESH,
            )
        pl.semaphore_wait(barrier_sem, 2)

        ag_ref[pl.ds(my * m_per, m_per), :] = a_ref[...]

        def compute_chunk(idx):
            stage_ref[...] = jnp.dot(
                ag_ref[pl.ds(idx * m_per, m_per), :], b_ref[...],
                preferred_element_type=jnp.float32,
            ).astype(jnp.bfloat16)
            copy = pltpu.make_async_copy(
                stage_ref, out_ref.at[pl.ds(idx * m_per, m_per), :], copy_sem
            )
            copy.start()
            copy.wait()

        def chunk_rdma(idx, send_sem, recv_sem, dev):
            return pltpu.make_async_remote_copy(
                src_ref=ag_ref.at[pl.ds(idx * m_per, m_per), :],
                dst_ref=ag_ref.at[pl.ds(idx * m_per, m_per), :],
                send_sem=send_sem,
                recv_sem=recv_sem,
                device_id=(dev,),
                device_id_type=pl.DeviceIdType.MESH,
            )

        for h in range(CW):
            cw_origin = lut(RING, lax.rem(p + N_DEV - h, N_DEV))
            cw_in = lut(RING, lax.rem(p + 2 * N_DEV - h - 1, N_DEV))
            send_cw = chunk_rdma(
                cw_origin, cw_send_sems.at[h], cw_recv_sems.at[h], right
            )
            send_cw.start()
            if h < CCW:
                ccw_origin = lut(RING, lax.rem(p + h, N_DEV))
                ccw_in = lut(RING, lax.rem(p + h + 1, N_DEV))
                send_ccw = chunk_rdma(
                    ccw_origin, ccw_send_sems.at[h], ccw_recv_sems.at[h], left
                )
                send_ccw.start()
            if h == 0:
                compute_chunk(my)
            else:
                compute_chunk(cw_origin)
                compute_chunk(lut(RING, lax.rem(p + h, N_DEV)))
            send_cw.wait_send()
            chunk_rdma(
                cw_in, cw_send_sems.at[h], cw_recv_sems.at[h], left
            ).wait_recv()
            if h < CCW:
                send_ccw.wait_send()
                chunk_rdma(
                    ccw_in, ccw_send_sems.at[h], ccw_recv_sems.at[h], right
                ).wait_recv()

        compute_chunk(lut(RING, lax.rem(p + N_DEV - CW, N_DEV)))

    return pl.pallas_call(
        body,
        out_shape=jax.ShapeDtypeStruct((N_DEV * m_per, n), jnp.bfloat16),
        in_specs=[
            pl.BlockSpec(memory_space=pltpu.VMEM),
            pl.BlockSpec(memory_space=pltpu.VMEM),
        ],
        out_specs=pl.BlockSpec(memory_space=pl.ANY),
        scratch_shapes=[
            pltpu.VMEM((N_DEV * m_per, k), jnp.bfloat16),
            pltpu.VMEM((m_per, n), jnp.bfloat16),
            pltpu.SemaphoreType.DMA((CW,)),
            pltpu.SemaphoreType.DMA((CW,)),
            pltpu.SemaphoreType.DMA((CCW,)),
            pltpu.SemaphoreType.DMA((CCW,)),
            pltpu.SemaphoreType.DMA,
        ],
        compiler_params=pltpu.CompilerParams(collective_id=0),
    )(A, B)
